# Initial kernel scaffold; baseline (speedup 1.0000x reference)
import jax
import jax.numpy as jnp
from jax.experimental import pallas as pl


def kernel(x, edge_index, edge_masks, ptr, prep_W1, prep_b1, prep_W2, prep_b2, msg_W1, msg_b1, msg_W2, msg_b2, upd_W1, upd_b1, upd_W2, upd_b2, dag_W1, dag_b1, dag_W2, dag_b2, glob_W1, glob_b1, glob_W2, glob_b2):
    raise NotImplementedError("write your pallas kernel here")


# speedup vs baseline: 4.1427x; 4.1427x over previous
import dataclasses
import functools

import jax
import jax.numpy as jnp
from jax import lax
from jax.experimental import pallas as pl
from jax.experimental.pallas import tpu as pltpu
from jax.experimental.pallas import tpu_sc as plsc

N = 10000
E = 320000
D = 128
NDAG = 128
NC = 2
NS = 16
EPS = E // NS
NITER = EPS // 16
HALF = N // NC
HPAD = 6144
STRIPE = HPAD // NS
CAP = 20224
TRASH = HALF
LVMUL = 16384
BSZ = 1024
NB = 5
BSHIFT = 10
BPAD = 1152
BCH = BPAD // 128
BTRASH = BSZ
BLK = 1000
GRID = N // BLK

_f32 = jnp.float32
_i32 = jnp.int32


def _sc_cparams():
    cp = pltpu.CompilerParams()
    if "needs_layout_passes" in pltpu.CompilerParams.__dataclass_fields__:
        cp = dataclasses.replace(cp, needs_layout_passes=False)
    return cp


def _deg_lv_body(row_hbm, em1_hbm, em2_hbm, em3_hbm, lv_hbm, deg_hbm,
                 vrow, ve1, ve2, ve3, vlv, crow, ridx, ones1, zcnt, deg_sh):
    core = lax.axis_index("c")
    sid = lax.axis_index("s")
    base = sid * EPS
    z16 = jnp.zeros((16,), _f32)
    o16 = jnp.ones((16,), _f32)

    @pl.loop(0, (4 * STRIPE) // 16)
    def _(r):
        zcnt[pl.ds(r * 16, 16)] = z16

    for k in range(8):
        ones1[pl.ds(k * 16, 16)] = o16

    pltpu.sync_copy(zcnt, deg_sh.at[pl.ds(sid * 4 * STRIPE, 4 * STRIPE)])

    pltpu.sync_copy(row_hbm.at[pl.ds(base, EPS)], vrow)
    pltpu.sync_copy(em1_hbm.at[pl.ds(base, EPS)], ve1)
    pltpu.sync_copy(em2_hbm.at[pl.ds(base, EPS)], ve2)
    pltpu.sync_copy(em3_hbm.at[pl.ds(base, EPS)], ve3)

    nlo = core * HALF

    def cbody(i, off):
        s = pl.ds(i * 16, 16)
        lv16 = ve1[s] + 2 * ve2[s] + 3 * ve3[s]
        vlv[s] = vrow[s] + LVMUL * lv16
        rl = vrow[s] - nlo
        m = (rl >= 0) & (rl < HALF)
        mi = m.astype(_i32)
        pos = off + jnp.cumsum(mi) - 1
        plsc.store_scatter(crow, [pos], rl + lv16 * HPAD, mask=m)
        return off + jnp.sum(mi)

    off = lax.fori_loop(0, NITER, cbody, _i32(0))

    iota16 = lax.iota(_i32, 16)
    trash16 = jnp.full((16,), TRASH, _i32)
    for k in range(8):
        plsc.store_scatter(crow, [off + iota16 + k * 16], trash16)

    plsc.subcore_barrier()

    nch = (off + 127) // 128

    def gbody(j, c):
        for k in range(8):
            ridx[pl.ds(k * 16, 16)] = crow[pl.ds(j * 128 + k * 16, 16)]
        pltpu.sync_copy(ones1, deg_sh.at[ridx], add=True)
        return c

    lax.fori_loop(0, nch, gbody, _i32(0))

    plsc.subcore_barrier()

    @pl.when(core == 0)
    def _():
        pltpu.sync_copy(vlv, lv_hbm.at[pl.ds(base, EPS)])

    pltpu.sync_copy(deg_sh.at[pl.ds(sid * 4 * STRIPE, 4 * STRIPE)],
                    deg_hbm.at[pl.ds(core * 4 * HPAD + sid * 4 * STRIPE,
                                     4 * STRIPE)])


def _sc_deg_lv(row, em1, em2, em3):
    mesh = plsc.VectorSubcoreMesh(core_axis_name="c", subcore_axis_name="s")
    return pl.kernel(
        _deg_lv_body,
        out_type=(jax.ShapeDtypeStruct((E,), _i32),
                  jax.ShapeDtypeStruct((NC * 4 * HPAD,), _f32)),
        mesh=mesh,
        compiler_params=_sc_cparams(),
        scratch_types=[
            pltpu.VMEM((EPS,), _i32),
            pltpu.VMEM((EPS,), _i32),
            pltpu.VMEM((EPS,), _i32),
            pltpu.VMEM((EPS,), _i32),
            pltpu.VMEM((EPS,), _i32),
            pltpu.VMEM((CAP,), _i32),
            pltpu.VMEM((128,), _i32),
            pltpu.VMEM((128,), _f32),
            pltpu.VMEM((4 * STRIPE,), _f32),
            pltpu.VMEM_SHARED((4 * HPAD,), _f32),
        ],
    )(row, em1, em2, em3)


def _level_body(lvl, lvr_hbm, col_hbm, msg_hbm, agg_hbm,
                vlr, vcol, crow, ccol, ridx, cidx, ridx2, cidx2, gbuf,
                gbuf2, gsem, gsem2, agg_sh):
    core = lax.axis_index("c")
    sid = lax.axis_index("s")
    base = sid * EPS
    z16 = jnp.zeros((16,), _f32)
    o16 = jnp.ones((16,), _f32)

    pltpu.sync_copy(lvr_hbm.at[pl.ds(base, EPS)], vlr)
    pltpu.sync_copy(col_hbm.at[pl.ds(base, EPS)], vcol)

    shift = lvl * LVMUL + core * HALF
    iota16 = lax.iota(_i32, 16)

    def c1body(i, off):
        sl = pl.ds(i * 16, 16)
        t = vlr[sl] - shift
        m0 = (t >= 0) & (t < HALF)
        mi = m0.astype(_i32)
        pos = off + jnp.cumsum(mi) - 1
        pk = t | (vcol[sl] << 13)
        plsc.store_scatter(crow, [pos], pk, mask=m0)
        return off + jnp.sum(mi)

    cnt1 = lax.fori_loop(0, NITER, c1body, _i32(0))

    plsc.store_scatter(crow, [cnt1 + iota16], jnp.full((16,), 7 * BSZ, _i32))
    nc1 = (cnt1 + 15) // 16

    def c2body(i, cnts):
        bk = (crow[pl.ds(i * 16, 16)] & 8191) >> BSHIFT
        return tuple(cnts[b] + jnp.sum((bk == b).astype(_i32))
                     for b in range(NB))

    cnts = lax.fori_loop(0, nc1, c2body, (0,) * NB)

    bases = [_i32(0)]
    for b in range(NB):
        seg = ((cnts[b] + 127) // 128) * 128 + 128
        bases.append(bases[b] + seg)

    def c3body(i, offs):
        v = crow[pl.ds(i * 16, 16)]
        bk = (v & 8191) >> BSHIFT
        new = []
        for b in range(NB):
            mb = bk == b
            mi = mb.astype(_i32)
            pos = bases[b] + offs[b] + jnp.cumsum(mi) - 1
            plsc.store_scatter(ccol, [pos], v, mask=mb)
            new.append(offs[b] + jnp.sum(mi))
        return tuple(new)

    lax.fori_loop(0, nc1, c3body, (_i32(0),) * NB)

    for b in range(NB):
        tr16 = jnp.full((16,), (b + 1) * BSZ, _i32)
        for k in range(8):
            plsc.store_scatter(ccol, [bases[b] + cnts[b] + iota16 + k * 16],
                               tr16)

    for b in range(NB):
        @pl.when(sid < BCH)
        def _():
            @pl.loop(0, 128)
            def _(r):
                for k in range(8):
                    gbuf[r, pl.ds(k * 16, 16)] = z16

            pltpu.sync_copy(gbuf.at[pl.ds(0, 128)],
                            agg_sh.at[pl.ds(sid * 128, 128)])

        plsc.subcore_barrier()

        nch = (cnts[b] + 127) // 128
        cb = bases[b]

        def stage(j, ci, ri):
            for k in range(8):
                v = ccol[pl.ds(cb + j * 128 + k * 16, 16)]
                ci[pl.ds(k * 16, 16)] = v >> 13
                ri[pl.ds(k * 16, 16)] = (v & 8191) - b * BSZ

        def pairbody(pi, c):
            j0 = pi * 2
            j1 = j0 + 1
            stage(j0, cidx, ridx)
            ga = pltpu.async_copy(msg_hbm.at[cidx], gbuf, gsem)

            @pl.when(j1 < nch)
            def _():
                stage(j1, cidx2, ridx2)
                pltpu.async_copy(msg_hbm.at[cidx2], gbuf2, gsem2)

            ga.wait()
            pltpu.sync_copy(gbuf, agg_sh.at[ridx], add=True)

            @pl.when(j1 < nch)
            def _():
                pltpu.make_async_copy(msg_hbm.at[cidx2], gbuf2, gsem2).wait()
                pltpu.sync_copy(gbuf2, agg_sh.at[ridx2], add=True)
            return c

        lax.fori_loop(0, (nch + 1) // 2, pairbody, _i32(0))

        plsc.subcore_barrier()

        @pl.when(sid < BCH)
        def _():
            ob = (core * NB + b) * BPAD + sid * 128
            pltpu.sync_copy(agg_sh.at[pl.ds(sid * 128, 128)],
                            agg_hbm.at[pl.ds(ob, 128)])


def _sc_level(lvl, lvr, col, msg):
    mesh = plsc.VectorSubcoreMesh(core_axis_name="c", subcore_axis_name="s")
    return pl.kernel(
        functools.partial(_level_body, lvl),
        out_type=jax.ShapeDtypeStruct((NC * NB * BPAD, D), _f32),
        mesh=mesh,
        compiler_params=_sc_cparams(),
        scratch_types=[
            pltpu.VMEM((EPS,), _i32),
            pltpu.VMEM((EPS,), _i32),
            pltpu.VMEM((CAP,), _i32),
            pltpu.VMEM((CAP,), _i32),
            pltpu.VMEM((128,), _i32),
            pltpu.VMEM((128,), _i32),
            pltpu.VMEM((128,), _i32),
            pltpu.VMEM((128,), _i32),
            pltpu.VMEM((128, D), _f32),
            pltpu.VMEM((128, D), _f32),
            pltpu.SemaphoreType.DMA,
            pltpu.SemaphoreType.DMA,
            pltpu.VMEM_SHARED((BPAD, D), _f32),
        ],
    )(lvr, col, msg)


def _mlp2(t, w1, b1, w2, b2):
    h = jnp.maximum(jnp.dot(t, w1) + b1, 0.0)
    return jnp.dot(h, w2) + b2


_w = pl.BlockSpec((D, D), lambda i: (0, 0))
_b = pl.BlockSpec((1, D), lambda i: (0, 0))
_rows = pl.BlockSpec((BLK, D), lambda i: (i, 0))
_col1 = pl.BlockSpec((BLK, 1), lambda i: (i, 0))
_mk = jax.ShapeDtypeStruct((N, D), _f32)


def _prep_body(x_ref, pw1, pb1, pw2, pb2, uw1, ub1, uw2, ub2, hi_ref, u0_ref):
    hi = _mlp2(x_ref[...], pw1[...], pb1[...], pw2[...], pb2[...])
    hi_ref[...] = hi
    u0_ref[...] = _mlp2(hi, uw1[...], ub1[...], uw2[...], ub2[...])


def _first_body(u0_ref, dg_ref, mw1, mb1, mw2, mb2, h_ref, msg_ref):
    h = jnp.where(dg_ref[...] == 0.0, u0_ref[...], 0.0)
    h_ref[...] = h
    msg_ref[...] = _mlp2(h, mw1[...], mb1[...], mw2[...], mb2[...])


def _mid_body(hp_ref, hi_ref, ag_ref, dc_ref,
              uw1, ub1, uw2, ub2, mw1, mb1, mw2, mb2, h_ref, msg_ref):
    upd = _mlp2(ag_ref[...], uw1[...], ub1[...], uw2[...], ub2[...])
    h = jnp.where(dc_ref[...] > 0.0, hi_ref[...] + upd, hp_ref[...])
    h_ref[...] = h
    msg_ref[...] = _mlp2(h, mw1[...], mb1[...], mw2[...], mb2[...])


def _last_body(hp_ref, hi_ref, ag_ref, dc_ref, uw1, ub1, uw2, ub2, h_ref):
    upd = _mlp2(ag_ref[...], uw1[...], ub1[...], uw2[...], ub2[...])
    h_ref[...] = jnp.where(dc_ref[...] > 0.0, hi_ref[...] + upd, hp_ref[...])


def _dag_body(x_ref, h_ref, lo_ref, hi_ref, dw1a, dw1b, db1, dw2, db2,
              gw1, gb1, gw2, gb2, hd_ref, hg_ref, acc):
    i = pl.program_id(0)
    t = jnp.maximum(jnp.dot(x_ref[...], dw1a[...]) +
                    jnp.dot(h_ref[...], dw1b[...]) + db1[...], 0.0)
    z = jnp.dot(t, dw2[...]) + db2[...]
    ids = lax.broadcasted_iota(_i32, (NDAG, BLK), 1) + i * BLK
    seg = ((ids >= lo_ref[...]) & (ids < hi_ref[...])).astype(_f32)

    @pl.when(i == 0)
    def _():
        acc[...] = jnp.zeros((NDAG, D), _f32)

    acc[...] += jnp.dot(seg, z)

    @pl.when(i == GRID - 1)
    def _():
        hd = acc[...]
        hd_ref[...] = hd
        g = _mlp2(hd, gw1[...], gb1[...], gw2[...], gb2[...])
        hg_ref[...] = jnp.sum(g, axis=0, keepdims=True)


def kernel(x, edge_index, edge_masks, ptr,
           prep_W1, prep_b1, prep_W2, prep_b2,
           msg_W1, msg_b1, msg_W2, msg_b2,
           upd_W1, upd_b1, upd_W2, upd_b2,
           dag_W1, dag_b1, dag_W2, dag_b2,
           glob_W1, glob_b1, glob_W2, glob_b2):
    row = edge_index[0].astype(_i32)
    col = edge_index[1].astype(_i32)
    emi = edge_masks.astype(_i32)

    pb1, pb2 = prep_b1.reshape(1, D), prep_b2.reshape(1, D)
    mb1, mb2 = msg_b1.reshape(1, D), msg_b2.reshape(1, D)
    ub1, ub2 = upd_b1.reshape(1, D), upd_b2.reshape(1, D)
    db1, db2 = dag_b1.reshape(1, D), dag_b2.reshape(1, D)
    gb1, gb2 = glob_b1.reshape(1, D), glob_b2.reshape(1, D)
    dW1a, dW1b = dag_W1[:D], dag_W1[D:]

    lvr, dall = _sc_deg_lv(row, emi[1], emi[2], emi[3])
    dall = dall.reshape(NC, 4, HPAD)[:, :, :HALF]
    dcnts = [dall[:, l].reshape(N, 1) for l in range(4)]
    deg = dcnts[0] + dcnts[1] + dcnts[2] + dcnts[3]

    h_init, u0 = pl.pallas_call(
        _prep_body,
        grid=(GRID,),
        in_specs=[_rows, _w, _b, _w, _b, _w, _b, _w, _b],
        out_specs=[_rows, _rows],
        out_shape=[_mk, _mk],
    )(x, prep_W1, pb1, prep_W2, pb2, upd_W1, ub1, upd_W2, ub2)

    h, msg = pl.pallas_call(
        _first_body,
        grid=(GRID,),
        in_specs=[_rows, _col1, _w, _b, _w, _b],
        out_specs=[_rows, _rows],
        out_shape=[_mk, _mk],
    )(u0, deg, msg_W1, mb1, msg_W2, mb2)

    for lvl in range(3, -1, -1):
        agg2 = _sc_level(lvl, lvr, col, msg)
        agg = agg2.reshape(NC, NB, BPAD, D)[:, :, :BSZ].reshape(
            NC, NB * BSZ, D)[:, :HALF].reshape(N, D)
        dcnt = dcnts[lvl]
        if lvl > 0:
            h, msg = pl.pallas_call(
                _mid_body,
                grid=(GRID,),
                in_specs=[_rows, _rows, _rows, _col1,
                          _w, _b, _w, _b, _w, _b, _w, _b],
                out_specs=[_rows, _rows],
                out_shape=[_mk, _mk],
            )(h, h_init, agg, dcnt,
              upd_W1, ub1, upd_W2, ub2, msg_W1, mb1, msg_W2, mb2)
        else:
            h = pl.pallas_call(
                _last_body,
                grid=(GRID,),
                in_specs=[_rows, _rows, _rows, _col1, _w, _b, _w, _b],
                out_specs=_rows,
                out_shape=_mk,
            )(h, h_init, agg, dcnt, upd_W1, ub1, upd_W2, ub2)

    lo = ptr[:NDAG].astype(_i32).reshape(NDAG, 1)
    hi = ptr[1:NDAG + 1].astype(_i32).reshape(NDAG, 1)
    h_dag, h_glob = pl.pallas_call(
        _dag_body,
        grid=(GRID,),
        in_specs=[_rows, _rows,
                  pl.BlockSpec((NDAG, 1), lambda i: (0, 0)),
                  pl.BlockSpec((NDAG, 1), lambda i: (0, 0)),
                  _w, _w, _b, _w, _b, _w, _b, _w, _b],
        out_specs=[pl.BlockSpec((NDAG, D), lambda i: (0, 0)),
                   pl.BlockSpec((1, D), lambda i: (0, 0))],
        out_shape=[jax.ShapeDtypeStruct((NDAG, D), _f32),
                   jax.ShapeDtypeStruct((1, D), _f32)],
        scratch_shapes=[pltpu.VMEM((NDAG, D), _f32)],
    )(x, h, lo, hi, dW1a, dW1b, db1, dag_W2, db2,
      glob_W1, gb1, glob_W2, gb2)

    return h, h_dag, h_glob
